# Initial kernel scaffold; baseline (speedup 1.0000x reference)
import jax
import jax.numpy as jnp
from jax.experimental import pallas as pl


def kernel(x, edge_index, W1l, b1, W1r, W2l, b2, W2r, W3l, b3, W3r, W4l, b4, W4r, W5l, b5, W5r):
    raise NotImplementedError("write your pallas kernel here")


# speedup vs baseline: 3.8579x; 3.8579x over previous
import functools

import jax
import jax.numpy as jnp
from jax import lax
from jax.experimental import pallas as pl
from jax.experimental.pallas import tpu as pltpu
from jax.experimental.pallas import tpu_sc as plsc

N = 10000
D = 128
E = 320000
NC = 2
NS = 16
NW = NC * NS
CHUNK = 128
T = -(-E // (NW * CHUNK))
E_PAD = NW * T * CHUNK
N_PAD = 10240
RPT = N_PAD // NS
ZROWS = 128

_MESH = dict(core_axis_name="c", subcore_axis_name="s")


def _fill(ref, rows, cols, vec):
    def row(i, c):
        for j in range(cols // 16):
            ref[i, pl.ds(j * 16, 16)] = vec
        return c
    lax.fori_loop(0, rows, row, 0)


def _make_sc_agg():
    def body(u_hbm, src_hbm, dst_hbm, out_hbm, src_v, dst_v, rows_v, agg_sh,
             sem):
        cid = lax.axis_index("c")
        sid = lax.axis_index("s")
        wid = sid * NC + cid
        _fill(rows_v, ZROWS, D, jnp.zeros((16,), jnp.float32))
        r0 = sid * RPT
        for k in range(RPT // ZROWS):
            pltpu.sync_copy(rows_v, agg_sh.at[pl.ds(r0 + k * ZROWS, ZROWS)])
        plsc.subcore_barrier()

        def step(i, c):
            base = (wid * T + i) * CHUNK
            pltpu.sync_copy(src_hbm.at[pl.ds(base, CHUNK)], src_v)
            pltpu.sync_copy(dst_hbm.at[pl.ds(base, CHUNK)], dst_v)
            pltpu.async_copy(u_hbm.at[src_v], rows_v, sem).wait()
            pltpu.sync_copy(rows_v, agg_sh.at[dst_v], add=True)
            return c
        lax.fori_loop(0, T, step, 0)
        plsc.subcore_barrier()
        for k in range(RPT // ZROWS):
            r = r0 + k * ZROWS
            pltpu.sync_copy(agg_sh.at[pl.ds(r, ZROWS)], rows_v)
            pltpu.sync_copy(rows_v, out_hbm.at[cid, pl.ds(r, ZROWS)])

    return pl.kernel(
        body,
        out_type=jax.ShapeDtypeStruct((NC, N_PAD, D), jnp.float32),
        scratch_types=[
            pltpu.VMEM((CHUNK,), jnp.int32),
            pltpu.VMEM((CHUNK,), jnp.int32),
            pltpu.VMEM((ZROWS, D), jnp.float32),
            pltpu.VMEM_SHARED((N_PAD, D), jnp.float32),
            pltpu.SemaphoreType.DMA,
        ],
        mesh=plsc.VectorSubcoreMesh(**_MESH),
    )


def _make_sc_deg():
    def body(dst_hbm, out_hbm, dst_v, ones_v, deg_sh):
        cid = lax.axis_index("c")
        sid = lax.axis_index("s")
        wid = sid * NC + cid
        _fill(ones_v, ZROWS, D, jnp.zeros((16,), jnp.float32))
        r0 = sid * RPT
        for k in range(RPT // ZROWS):
            pltpu.sync_copy(ones_v, deg_sh.at[pl.ds(r0 + k * ZROWS, ZROWS)])
        _fill(ones_v, CHUNK, D, jnp.ones((16,), jnp.float32))
        plsc.subcore_barrier()

        def step(i, c):
            base = (wid * T + i) * CHUNK
            pltpu.sync_copy(dst_hbm.at[pl.ds(base, CHUNK)], dst_v)
            pltpu.sync_copy(ones_v, deg_sh.at[dst_v], add=True)
            return c
        lax.fori_loop(0, T, step, 0)
        plsc.subcore_barrier()
        for k in range(RPT // ZROWS):
            r = r0 + k * ZROWS
            pltpu.sync_copy(deg_sh.at[pl.ds(r, ZROWS)], ones_v)
            pltpu.sync_copy(ones_v, out_hbm.at[cid, pl.ds(r, ZROWS)])

    return pl.kernel(
        body,
        out_type=jax.ShapeDtypeStruct((NC, N_PAD, D), jnp.float32),
        scratch_types=[
            pltpu.VMEM((CHUNK,), jnp.int32),
            pltpu.VMEM((ZROWS, D), jnp.float32),
            pltpu.VMEM_SHARED((N_PAD, D), jnp.float32),
        ],
        mesh=plsc.VectorSubcoreMesh(**_MESH),
    )


_SC_AGG = _make_sc_agg()
_SC_DEG = _make_sc_deg()

BN = 2000
GRID = N // BN


def _tc_pre(x, wlT, wrT, b):
    def body(x_ref, wl_ref, wr_ref, b_ref, u_ref, v_ref):
        xb = x_ref[...]
        u_ref[...] = jnp.dot(xb, wl_ref[...], preferred_element_type=jnp.float32)
        v_ref[...] = jnp.dot(xb, wr_ref[...], preferred_element_type=jnp.float32) + b_ref[...]
    return pl.pallas_call(
        body,
        grid=(GRID,),
        in_specs=[
            pl.BlockSpec((BN, D), lambda i: (i, 0)),
            pl.BlockSpec((D, D), lambda i: (0, 0)),
            pl.BlockSpec((D, D), lambda i: (0, 0)),
            pl.BlockSpec((1, D), lambda i: (0, 0)),
        ],
        out_specs=[
            pl.BlockSpec((BN, D), lambda i: (i, 0)),
            pl.BlockSpec((BN, D), lambda i: (i, 0)),
        ],
        out_shape=[jax.ShapeDtypeStruct((N, D), jnp.float32)] * 2,
    )(x, wlT, wrT, b)


def _combine(p_ref, dg_ref, v_ref):
    ps = p_ref[0] + p_ref[1]
    dg = dg_ref[0, :, 0:1] + dg_ref[1, :, 0:1]
    inv = 1.0 / jnp.maximum(dg, 1.0)
    return jnp.maximum(ps * inv + v_ref[...], 0.0)


def _tc_mid(p, degp, v, wlT, wrT, b):
    def body(p_ref, dg_ref, v_ref, wl_ref, wr_ref, b_ref, u_ref, v2_ref):
        h = _combine(p_ref, dg_ref, v_ref)
        u_ref[...] = jnp.dot(h, wl_ref[...], preferred_element_type=jnp.float32)
        v2_ref[...] = jnp.dot(h, wr_ref[...], preferred_element_type=jnp.float32) + b_ref[...]
    return pl.pallas_call(
        body,
        grid=(GRID,),
        in_specs=[
            pl.BlockSpec((NC, BN, D), lambda i: (0, i, 0)),
            pl.BlockSpec((NC, BN, D), lambda i: (0, i, 0)),
            pl.BlockSpec((BN, D), lambda i: (i, 0)),
            pl.BlockSpec((D, D), lambda i: (0, 0)),
            pl.BlockSpec((D, D), lambda i: (0, 0)),
            pl.BlockSpec((1, D), lambda i: (0, 0)),
        ],
        out_specs=[
            pl.BlockSpec((BN, D), lambda i: (i, 0)),
            pl.BlockSpec((BN, D), lambda i: (i, 0)),
        ],
        out_shape=[jax.ShapeDtypeStruct((N, D), jnp.float32)] * 2,
    )(p, degp, v, wlT, wrT, b)


def _tc_final(p, degp, v):
    def body(p_ref, dg_ref, v_ref, o_ref):
        h = _combine(p_ref, dg_ref, v_ref)
        o_ref[...] = h[:, :17]
    return pl.pallas_call(
        body,
        grid=(GRID,),
        in_specs=[
            pl.BlockSpec((NC, BN, D), lambda i: (0, i, 0)),
            pl.BlockSpec((NC, BN, D), lambda i: (0, i, 0)),
            pl.BlockSpec((BN, D), lambda i: (i, 0)),
        ],
        out_specs=pl.BlockSpec((BN, 17), lambda i: (i, 0)),
        out_shape=jax.ShapeDtypeStruct((N, 17), jnp.float32),
    )(p, degp, v)


def kernel(x, edge_index, W1l, b1, W1r, W2l, b2, W2r, W3l, b3, W3r,
           W4l, b4, W4r, W5l, b5, W5r):
    pad = E_PAD - E
    srcp = jnp.concatenate([edge_index[0], jnp.zeros((pad,), jnp.int32)])
    dstp = jnp.concatenate([edge_index[1], jnp.full((pad,), N, jnp.int32)])

    degp = _SC_DEG(dstp)

    w5lT = jnp.zeros((D, D), jnp.float32).at[:, :17].set(W5l.T)
    w5rT = jnp.zeros((D, D), jnp.float32).at[:, :17].set(W5r.T)
    b5p = jnp.zeros((1, D), jnp.float32).at[0, :17].set(b5)

    u, v = _tc_pre(x, W1l.T, W1r.T, b1.reshape(1, D))
    for wl, wr, b in ((W2l, W2r, b2), (W3l, W3r, b3), (W4l, W4r, b4)):
        p = _SC_AGG(u, srcp, dstp)
        u, v = _tc_mid(p, degp, v, wl.T, wr.T, b.reshape(1, D))
    p = _SC_AGG(u, srcp, dstp)
    u, v = _tc_mid(p, degp, v, w5lT, w5rT, b5p)
    p = _SC_AGG(u, srcp, dstp)
    return _tc_final(p, degp, v)
